# baseline (device time: 273644 ns/iter reference)
import jax
import jax.numpy as jnp
from jax import lax
from jax.experimental import pallas as pl
from jax.experimental.pallas import tpu as pltpu

N_DEV = 8
M, N, K = 4096, 2048, 512
ENGINE_ROWS = (1408, 1344, 1344)
ENGINE_BASE = (0, 1408, 2752)
MASKS = ((1, 3, 4), (3, 4, 1), (4, 1, 3))
_MESH = pl.DeviceIdType.MESH


def _keep_bit(my, m):
    if m == 1:
        return (my ^ (my >> 1)) & 1
    if m == 3:
        return (my >> 1) & 1
    return (my >> 2) & 1


def _ar_body(x_ref, w_ref, sx_ref, sw_ref, out_ref, buf0, buf1, buf2,
             rs_send, rs_recv, ag_send, ag_recv, credits):
    bufs = (buf0, buf1, buf2)
    my = lax.axis_index("i")

    barrier = pltpu.get_barrier_semaphore()
    for m in (1, 3, 4):
        pl.semaphore_signal(
            barrier, inc=1, device_id=(my ^ m,), device_id_type=_MESH,
        )
    pl.semaphore_wait(barrier, 3)

    S, KB = [], []
    for e in range(3):
        s, kb = [ENGINE_BASE[e]], []
        for r in range(3):
            b = _keep_bit(my, MASKS[e][r])
            kb.append(b)
            s.append(s[-1] + b * (ENGINE_ROWS[e] >> (r + 1)))
        S.append(s)
        KB.append(kb)

    scale = sx_ref[0] * sw_ref[0]
    w = w_ref[...].astype(jnp.bfloat16) * scale.astype(jnp.bfloat16)

    def gemm_block(base, nrows):
        rows = pl.ds(base, nrows)
        xb = x_ref[rows, :].astype(jnp.bfloat16)
        out_ref[rows, :] = lax.dot_general(
            xb, w, (((1,), (0,)), ((), ())),
            preferred_element_type=jnp.float32,
        )

    def gemm_half(e, kept):
        H = ENGINE_ROWS[e] >> 1
        lo = (KB[e][0] == 0) if kept else (KB[e][0] == 1)

        @pl.when(lo)
        def _():
            gemm_block(ENGINE_BASE[e], H)

        @pl.when(jnp.logical_not(lo))
        def _():
            gemm_block(ENGINE_BASE[e] + H, H)

    def start_sends(e, r, n_sub):
        H = ENGINE_ROWS[e] >> (r + 1)
        Hs = H // n_sub
        partner = my ^ MASKS[e][r]
        send_base = S[e][r] + (1 - KB[e][r]) * H
        subs = []
        for j in range(n_sub):
            rd = pltpu.make_async_remote_copy(
                src_ref=out_ref.at[pl.ds(send_base + j * Hs, Hs), :],
                dst_ref=bufs[e].at[pl.ds(j * Hs, Hs), :],
                send_sem=rs_send.at[e, r, j],
                recv_sem=rs_recv.at[e, r, j],
                device_id=(partner,),
                device_id_type=_MESH,
            )
            rd.start()
            subs.append(rd)
        return subs

    for r in range(3):
        n_sub = (4, 2, 1)[r]
        rdmas = []
        for e in range(3):
            if r == 0:
                H = ENGINE_ROWS[e] >> 1
                Hq = H >> 2
                send_base = S[e][0] + (1 - KB[e][0]) * H
                partner = my ^ MASKS[e][0]
                lo = KB[e][0] == 1
                subs = []
                for q in range(4):
                    @pl.when(lo)
                    def _(q=q):
                        gemm_block(ENGINE_BASE[e] + q * Hq, Hq)

                    @pl.when(jnp.logical_not(lo))
                    def _(q=q):
                        gemm_block(ENGINE_BASE[e] + H + q * Hq, Hq)

                    rd = pltpu.make_async_remote_copy(
                        src_ref=out_ref.at[pl.ds(send_base + q * Hq, Hq), :],
                        dst_ref=bufs[e].at[pl.ds(q * Hq, Hq), :],
                        send_sem=rs_send.at[e, 0, q],
                        recv_sem=rs_recv.at[e, 0, q],
                        device_id=(partner,),
                        device_id_type=_MESH,
                    )
                    rd.start()
                    subs.append(rd)
                rdmas.append(subs)
            else:
                pl.semaphore_wait(credits.at[e], 1)
                rdmas.append(start_sends(e, r, n_sub))
        if r == 0:
            for e in range(3):
                gemm_half(e, kept=True)
        for j in range(n_sub):
            for e in range(3):
                Hs = (ENGINE_ROWS[e] >> (r + 1)) // n_sub
                rdmas[e][j].wait_recv()
                rows = pl.ds(S[e][r + 1] + j * Hs, Hs)
                out_ref[rows, :] += bufs[e][pl.ds(j * Hs, Hs), :]
        for e in range(3):
            for j in range(n_sub):
                rdmas[e][j].wait_send()
            if r < 2:
                pl.semaphore_signal(
                    credits.at[e], inc=1,
                    device_id=(my ^ MASKS[e][r + 1],), device_id_type=_MESH,
                )

    for k in (2, 1, 0):
        rdmas = []
        for e in range(3):
            L = ENGINE_ROWS[e] >> (k + 1)
            rows = pl.ds(S[e][k + 1], L)
            rd = pltpu.make_async_remote_copy(
                src_ref=out_ref.at[rows, :],
                dst_ref=out_ref.at[rows, :],
                send_sem=ag_send.at[e, k],
                recv_sem=ag_recv.at[e, k],
                device_id=(my ^ MASKS[e][k],),
                device_id_type=_MESH,
            )
            rd.start()
            rdmas.append(rd)
        for e in range(3):
            rdmas[e].wait()


def kernel(x, w_mat, scale_x, scale_w):
    return pl.pallas_call(
        _ar_body,
        out_shape=jax.ShapeDtypeStruct((M, N), jnp.float32),
        in_specs=[
            pl.BlockSpec(memory_space=pltpu.VMEM),
            pl.BlockSpec(memory_space=pltpu.VMEM),
            pl.BlockSpec(memory_space=pltpu.SMEM),
            pl.BlockSpec(memory_space=pltpu.SMEM),
        ],
        out_specs=pl.BlockSpec(memory_space=pltpu.VMEM),
        scratch_shapes=[
            pltpu.VMEM((ENGINE_ROWS[0] // 2, N), jnp.float32),
            pltpu.VMEM((ENGINE_ROWS[1] // 2, N), jnp.float32),
            pltpu.VMEM((ENGINE_ROWS[2] // 2, N), jnp.float32),
            pltpu.SemaphoreType.DMA((3, 3, 4)),
            pltpu.SemaphoreType.DMA((3, 3, 4)),
            pltpu.SemaphoreType.DMA((3, 3)),
            pltpu.SemaphoreType.DMA((3, 3)),
            pltpu.SemaphoreType.REGULAR((3,)),
        ],
        compiler_params=pltpu.CompilerParams(
            collective_id=0,
            vmem_limit_bytes=60 * 1024 * 1024,
        ),
    )(x, w_mat, scale_x, scale_w)


# device time: 265500 ns/iter; 1.0307x vs baseline; 1.0307x over previous
import jax
import jax.numpy as jnp
from jax import lax
from jax.experimental import pallas as pl
from jax.experimental.pallas import tpu as pltpu

N_DEV = 8
M, N, K = 4096, 2048, 512
ENGINE_ROWS = (1408, 1344, 1344)
ENGINE_BASE = (0, 1408, 2752)
MASKS = ((1, 3, 4), (3, 4, 1), (4, 1, 3))
_MESH = pl.DeviceIdType.MESH


def _keep_bit(my, m):
    if m == 1:
        return (my ^ (my >> 1)) & 1
    if m == 3:
        return (my >> 1) & 1
    return (my >> 2) & 1


def _ar_body(x_ref, w_ref, sx_ref, sw_ref, out_ref, acc, buf0, buf1, buf2,
             rs_send, rs_recv, ag_send, ag_recv, credits, fin_sems):
    bufs = (buf0, buf1, buf2)
    my = lax.axis_index("i")

    barrier = pltpu.get_barrier_semaphore()
    for m in (1, 3, 4):
        pl.semaphore_signal(
            barrier, inc=1, device_id=(my ^ m,), device_id_type=_MESH,
        )
    pl.semaphore_wait(barrier, 3)

    S, KB = [], []
    for e in range(3):
        s, kb = [ENGINE_BASE[e]], []
        for r in range(3):
            b = _keep_bit(my, MASKS[e][r])
            kb.append(b)
            s.append(s[-1] + b * (ENGINE_ROWS[e] >> (r + 1)))
        S.append(s)
        KB.append(kb)

    scale = sx_ref[0] * sw_ref[0]
    w = w_ref[...].astype(jnp.bfloat16) * scale.astype(jnp.bfloat16)

    def gemm_block(base, nrows):
        rows = pl.ds(base, nrows)
        xb = x_ref[rows, :].astype(jnp.bfloat16)
        acc[rows, :] = lax.dot_general(
            xb, w, (((1,), (0,)), ((), ())),
            preferred_element_type=jnp.float32,
        )

    def gemm_half(e, kept):
        H = ENGINE_ROWS[e] >> 1
        lo = (KB[e][0] == 0) if kept else (KB[e][0] == 1)

        @pl.when(lo)
        def _():
            gemm_block(ENGINE_BASE[e], H)

        @pl.when(jnp.logical_not(lo))
        def _():
            gemm_block(ENGINE_BASE[e] + H, H)

    def start_sends(e, r, n_sub):
        H = ENGINE_ROWS[e] >> (r + 1)
        Hs = H // n_sub
        partner = my ^ MASKS[e][r]
        send_base = S[e][r] + (1 - KB[e][r]) * H
        subs = []
        for j in range(n_sub):
            rd = pltpu.make_async_remote_copy(
                src_ref=acc.at[pl.ds(send_base + j * Hs, Hs), :],
                dst_ref=bufs[e].at[pl.ds(j * Hs, Hs), :],
                send_sem=rs_send.at[e, r, j],
                recv_sem=rs_recv.at[e, r, j],
                device_id=(partner,),
                device_id_type=_MESH,
            )
            rd.start()
            subs.append(rd)
        return subs

    import contextlib
    scope = jax.named_scope
    for r in range(3):
        n_sub = (4, 2, 1)[r]
        rdmas = []
        ctx = scope(f"issue_r{r}")
        ctx.__enter__()
        for e in range(3):
            if r == 0:
                H = ENGINE_ROWS[e] >> 1
                Hq = H >> 2
                send_base = S[e][0] + (1 - KB[e][0]) * H
                partner = my ^ MASKS[e][0]
                lo = KB[e][0] == 1
                subs = []
                for q in range(4):
                    @pl.when(lo)
                    def _(q=q):
                        gemm_block(ENGINE_BASE[e] + q * Hq, Hq)

                    @pl.when(jnp.logical_not(lo))
                    def _(q=q):
                        gemm_block(ENGINE_BASE[e] + H + q * Hq, Hq)

                    rd = pltpu.make_async_remote_copy(
                        src_ref=acc.at[pl.ds(send_base + q * Hq, Hq), :],
                        dst_ref=bufs[e].at[pl.ds(q * Hq, Hq), :],
                        send_sem=rs_send.at[e, 0, q],
                        recv_sem=rs_recv.at[e, 0, q],
                        device_id=(partner,),
                        device_id_type=_MESH,
                    )
                    rd.start()
                    subs.append(rd)
                rdmas.append(subs)
            else:
                pl.semaphore_wait(credits.at[e], 1)
                rdmas.append(start_sends(e, r, n_sub))
        ctx.__exit__(None, None, None)
        if r == 0:
            with scope("gemm_kept"):
                for e in range(3):
                    gemm_half(e, kept=True)
        with scope(f"consume_r{r}"):
            for j in range(n_sub):
                for e in range(3):
                    Hs = (ENGINE_ROWS[e] >> (r + 1)) // n_sub
                    rdmas[e][j].wait_recv()
                    rows = pl.ds(S[e][r + 1] + j * Hs, Hs)
                    acc[rows, :] += bufs[e][pl.ds(j * Hs, Hs), :]
            for e in range(3):
                for j in range(n_sub):
                    rdmas[e][j].wait_send()
                if r < 2:
                    pl.semaphore_signal(
                        credits.at[e], inc=1,
                        device_id=(my ^ MASKS[e][r + 1],),
                        device_id_type=_MESH,
                    )

    flushes = []
    for e in range(3):
        rows = pl.ds(S[e][3], ENGINE_ROWS[e] >> 3)
        cp = pltpu.make_async_copy(acc.at[rows, :], out_ref.at[rows, :],
                                   fin_sems.at[e])
        cp.start()
        flushes.append(cp)
    for cp in flushes:
        cp.wait()

    for k in (2, 1, 0):
        ag_ctx = jax.named_scope(f"ag_k{k}")
        ag_ctx.__enter__()
        rdmas = []
        for e in range(3):
            L = ENGINE_ROWS[e] >> (k + 1)
            rows = pl.ds(S[e][k + 1], L)
            rd = pltpu.make_async_remote_copy(
                src_ref=out_ref.at[rows, :],
                dst_ref=out_ref.at[rows, :],
                send_sem=ag_send.at[e, k],
                recv_sem=ag_recv.at[e, k],
                device_id=(my ^ MASKS[e][k],),
                device_id_type=_MESH,
            )
            rd.start()
            rdmas.append(rd)
        for e in range(3):
            rdmas[e].wait()
        ag_ctx.__exit__(None, None, None)


def kernel(x, w_mat, scale_x, scale_w):
    return pl.pallas_call(
        _ar_body,
        out_shape=jax.ShapeDtypeStruct((M, N), jnp.float32),
        in_specs=[
            pl.BlockSpec(memory_space=pltpu.VMEM),
            pl.BlockSpec(memory_space=pltpu.VMEM),
            pl.BlockSpec(memory_space=pltpu.SMEM),
            pl.BlockSpec(memory_space=pltpu.SMEM),
        ],
        out_specs=pl.BlockSpec(memory_space=pl.MemorySpace.ANY),
        scratch_shapes=[
            pltpu.VMEM((M, N), jnp.float32),
            pltpu.VMEM((ENGINE_ROWS[0] // 2, N), jnp.float32),
            pltpu.VMEM((ENGINE_ROWS[1] // 2, N), jnp.float32),
            pltpu.VMEM((ENGINE_ROWS[2] // 2, N), jnp.float32),
            pltpu.SemaphoreType.DMA((3, 3, 4)),
            pltpu.SemaphoreType.DMA((3, 3, 4)),
            pltpu.SemaphoreType.DMA((3, 3)),
            pltpu.SemaphoreType.DMA((3, 3)),
            pltpu.SemaphoreType.REGULAR((3,)),
            pltpu.SemaphoreType.DMA((3,)),
        ],
        compiler_params=pltpu.CompilerParams(
            collective_id=0,
            vmem_limit_bytes=60 * 1024 * 1024,
        ),
    )(x, w_mat, scale_x, scale_w)


# device time: 259791 ns/iter; 1.0533x vs baseline; 1.0220x over previous
import jax
import jax.numpy as jnp
from jax import lax
from jax.experimental import pallas as pl
from jax.experimental.pallas import tpu as pltpu

N_DEV = 8
M, N, K = 4096, 2048, 512
ENGINE_ROWS = (1408, 1344, 1344)
ENGINE_BASE = (0, 1408, 2752)
MASKS = ((1, 3, 4), (3, 4, 1), (4, 1, 3))
_MESH = pl.DeviceIdType.MESH


def _keep_bit(my, m):
    if m == 1:
        return (my ^ (my >> 1)) & 1
    if m == 3:
        return (my >> 1) & 1
    return (my >> 2) & 1


def _ar_body(x_ref, w_ref, sx_ref, sw_ref, out_ref, acc, buf0, buf1, buf2,
             rs_send, rs_recv, ag_send, ag_recv, credits, fin_sems):
    bufs = (buf0, buf1, buf2)
    my = lax.axis_index("i")

    barrier = pltpu.get_barrier_semaphore()
    for m in (1, 3, 4):
        pl.semaphore_signal(
            barrier, inc=1, device_id=(my ^ m,), device_id_type=_MESH,
        )
    pl.semaphore_wait(barrier, 3)

    S, KB = [], []
    for e in range(3):
        s, kb = [ENGINE_BASE[e]], []
        for r in range(3):
            b = _keep_bit(my, MASKS[e][r])
            kb.append(b)
            s.append(s[-1] + b * (ENGINE_ROWS[e] >> (r + 1)))
        S.append(s)
        KB.append(kb)

    scale = sx_ref[0] * sw_ref[0]
    w = w_ref[...].astype(jnp.bfloat16) * scale.astype(jnp.bfloat16)

    def gemm_block(base, nrows):
        rows = pl.ds(base, nrows)
        xb = x_ref[rows, :].astype(jnp.bfloat16)
        acc[rows, :] = lax.dot_general(
            xb, w, (((1,), (0,)), ((), ())),
            preferred_element_type=jnp.float32,
        )

    def gemm_half(e, kept):
        H = ENGINE_ROWS[e] >> 1
        lo = (KB[e][0] == 0) if kept else (KB[e][0] == 1)

        @pl.when(lo)
        def _():
            gemm_block(ENGINE_BASE[e], H)

        @pl.when(jnp.logical_not(lo))
        def _():
            gemm_block(ENGINE_BASE[e] + H, H)

    def start_sends(e, r, n_sub):
        H = ENGINE_ROWS[e] >> (r + 1)
        Hs = H // n_sub
        partner = my ^ MASKS[e][r]
        send_base = S[e][r] + (1 - KB[e][r]) * H
        subs = []
        for j in range(n_sub):
            rd = pltpu.make_async_remote_copy(
                src_ref=acc.at[pl.ds(send_base + j * Hs, Hs), :],
                dst_ref=bufs[e].at[pl.ds(j * Hs, Hs), :],
                send_sem=rs_send.at[e, r, j],
                recv_sem=rs_recv.at[e, r, j],
                device_id=(partner,),
                device_id_type=_MESH,
            )
            rd.start()
            subs.append(rd)
        return subs

    import contextlib
    scope = jax.named_scope
    for r in range(3):
        n_sub = (4, 2, 1)[r]
        rdmas = []
        ctx = scope(f"issue_r{r}")
        ctx.__enter__()
        if r == 0:
            rdmas = [[], [], []]
            for q in range(4):
                for e in range(3):
                    H = ENGINE_ROWS[e] >> 1
                    Hq = H >> 2
                    send_base = S[e][0] + (1 - KB[e][0]) * H
                    lo = KB[e][0] == 1

                    @pl.when(lo)
                    def _(e=e, q=q, Hq=Hq):
                        gemm_block(ENGINE_BASE[e] + q * Hq, Hq)

                    @pl.when(jnp.logical_not(lo))
                    def _(e=e, q=q, H=H, Hq=Hq):
                        gemm_block(ENGINE_BASE[e] + H + q * Hq, Hq)

                    rd = pltpu.make_async_remote_copy(
                        src_ref=acc.at[pl.ds(send_base + q * Hq, Hq), :],
                        dst_ref=bufs[e].at[pl.ds(q * Hq, Hq), :],
                        send_sem=rs_send.at[e, 0, q],
                        recv_sem=rs_recv.at[e, 0, q],
                        device_id=(my ^ MASKS[e][0],),
                        device_id_type=_MESH,
                    )
                    rd.start()
                    rdmas[e].append(rd)
        else:
            for e in range(3):
                pl.semaphore_wait(credits.at[e], 1)
                rdmas.append(start_sends(e, r, n_sub))
        ctx.__exit__(None, None, None)
        if r == 0:
            with scope("gemm_kept"):
                for e in range(3):
                    gemm_half(e, kept=True)
        with scope(f"consume_r{r}"):
            for j in range(n_sub):
                for e in range(3):
                    Hs = (ENGINE_ROWS[e] >> (r + 1)) // n_sub
                    rdmas[e][j].wait_recv()
                    rows = pl.ds(S[e][r + 1] + j * Hs, Hs)
                    acc[rows, :] += bufs[e][pl.ds(j * Hs, Hs), :]
            for e in range(3):
                for j in range(n_sub):
                    rdmas[e][j].wait_send()
                if r < 2:
                    pl.semaphore_signal(
                        credits.at[e], inc=1,
                        device_id=(my ^ MASKS[e][r + 1],),
                        device_id_type=_MESH,
                    )

    flushes = []
    for e in range(3):
        rows = pl.ds(S[e][3], ENGINE_ROWS[e] >> 3)
        cp = pltpu.make_async_copy(acc.at[rows, :], out_ref.at[rows, :],
                                   fin_sems.at[e])
        cp.start()
        flushes.append(cp)

    for k in (2, 1, 0):
        ag_ctx = jax.named_scope(f"ag_k{k}")
        ag_ctx.__enter__()
        if k == 1:
            for cp in flushes:
                cp.wait()
        rdmas = []
        for e in range(3):
            L = ENGINE_ROWS[e] >> (k + 1)
            rows = pl.ds(S[e][k + 1], L)
            src = acc if k == 2 else out_ref
            rd = pltpu.make_async_remote_copy(
                src_ref=src.at[rows, :],
                dst_ref=out_ref.at[rows, :],
                send_sem=ag_send.at[e, k],
                recv_sem=ag_recv.at[e, k],
                device_id=(my ^ MASKS[e][k],),
                device_id_type=_MESH,
            )
            rd.start()
            rdmas.append(rd)
        for e in range(3):
            rdmas[e].wait()
        ag_ctx.__exit__(None, None, None)


def kernel(x, w_mat, scale_x, scale_w):
    return pl.pallas_call(
        _ar_body,
        out_shape=jax.ShapeDtypeStruct((M, N), jnp.float32),
        in_specs=[
            pl.BlockSpec(memory_space=pltpu.VMEM),
            pl.BlockSpec(memory_space=pltpu.VMEM),
            pl.BlockSpec(memory_space=pltpu.SMEM),
            pl.BlockSpec(memory_space=pltpu.SMEM),
        ],
        out_specs=pl.BlockSpec(memory_space=pl.MemorySpace.ANY),
        scratch_shapes=[
            pltpu.VMEM((M, N), jnp.float32),
            pltpu.VMEM((ENGINE_ROWS[0] // 2, N), jnp.float32),
            pltpu.VMEM((ENGINE_ROWS[1] // 2, N), jnp.float32),
            pltpu.VMEM((ENGINE_ROWS[2] // 2, N), jnp.float32),
            pltpu.SemaphoreType.DMA((3, 3, 4)),
            pltpu.SemaphoreType.DMA((3, 3, 4)),
            pltpu.SemaphoreType.DMA((3, 3)),
            pltpu.SemaphoreType.DMA((3, 3)),
            pltpu.SemaphoreType.REGULAR((3,)),
            pltpu.SemaphoreType.DMA((3,)),
        ],
        compiler_params=pltpu.CompilerParams(
            collective_id=0,
            vmem_limit_bytes=60 * 1024 * 1024,
        ),
    )(x, w_mat, scale_x, scale_w)
